# baseline (speedup 1.0000x reference)
import jax
import jax.numpy as jnp
from jax import lax
from jax.experimental import pallas as pl
from jax.experimental.pallas import tpu as pltpu
from jax.experimental.pallas import tpu_sc as plsc

_R = 128
_B, _S, _L = 4, 8192, 2048
_WL = 128
_NC = _S // _R
_NV = _WL // 16


def _sc_body(x_hbm, out_hbm, ib0, ib1, ob0, ob1, si0, si1, so0, so1):
    wid = lax.axis_index("s") * 2 + lax.axis_index("c")
    lane0 = (wid % 16) * _WL
    b0 = (wid // 16) * 2

    def in_cp(b, ci, buf, sem):
        return pltpu.make_async_copy(
            x_hbm.at[b, pl.ds(ci * _R, _R), pl.ds(lane0, _WL)], buf, sem
        )

    def out_cp(b, ci, buf, sem):
        return pltpu.make_async_copy(
            buf, out_hbm.at[b, pl.ds(ci * _R, _R), pl.ds(lane0, _WL)], sem
        )

    def scan_chunk(ibuf, obuf, accs):
        def row4(q, accs):
            for rr in range(4):
                r = q * 4 + rr
                new = []
                for c in range(_NV):
                    a = accs[c] + ibuf[r, pl.ds(c * 16, 16)]
                    obuf[r, pl.ds(c * 16, 16)] = a
                    new.append(a)
                accs = tuple(new)
            return accs

        return lax.fori_loop(0, _R // 4, row4, accs)

    for bi in range(2):
        b = b0 + bi
        in_cp(b, 0, ib0, si0).start()

        def pair(i, accs):
            c0 = 2 * i
            in_cp(b, c0 + 1, ib1, si1).start()
            in_cp(b, c0, ib0, si0).wait()

            @pl.when(i > 0)
            def _():
                out_cp(b, c0, ob0, so0).wait()

            accs = scan_chunk(ib0, ob0, accs)
            out_cp(b, c0, ob0, so0).start()
            nxt = jnp.minimum(c0 + 2, _NC - 1)
            in_cp(b, nxt, ib0, si0).start()
            in_cp(b, c0 + 1, ib1, si1).wait()

            @pl.when(i > 0)
            def _():
                out_cp(b, c0 + 1, ob1, so1).wait()

            accs = scan_chunk(ib1, ob1, accs)
            out_cp(b, c0 + 1, ob1, so1).start()
            return accs

        zeros = tuple(jnp.zeros((16,), jnp.float32) for _ in range(_NV))
        lax.fori_loop(0, _NC // 2, pair, zeros)
        in_cp(b, _NC - 1, ib0, si0).wait()
        out_cp(b, _NC - 2, ob0, so0).wait()
        out_cp(b, _NC - 1, ob1, so1).wait()


def kernel(x):
    k = pl.kernel(
        _sc_body,
        out_type=jax.ShapeDtypeStruct((_B, _S, _L), jnp.float32),
        mesh=plsc.VectorSubcoreMesh(core_axis_name="c", subcore_axis_name="s"),
        scratch_types=[
            pltpu.VMEM((_R, _WL), jnp.float32),
            pltpu.VMEM((_R, _WL), jnp.float32),
            pltpu.VMEM((_R, _WL), jnp.float32),
            pltpu.VMEM((_R, _WL), jnp.float32),
            pltpu.SemaphoreType.DMA,
            pltpu.SemaphoreType.DMA,
            pltpu.SemaphoreType.DMA,
            pltpu.SemaphoreType.DMA,
        ],
    )
    return k(x)


# speedup vs baseline: 1.1850x; 1.1850x over previous
import jax
import jax.numpy as jnp
from jax.experimental import pallas as pl
from jax.experimental.pallas import tpu as pltpu

_BS = 256
_U = 2
_NB = 4


def _scan_body(x_ref, o_ref, carry_ref):
    j = pl.program_id(1)

    @pl.when(j == 0)
    def _():
        carry_ref[...] = jnp.zeros_like(carry_ref)

    L = x_ref.shape[2]
    rows = 8 * _U

    def group(k, carry):
        newc = []
        for n in range(_NB):
            vs = []
            for u in range(_U):
                v = x_ref[n, pl.ds(k * rows + u * 8, 8), :]
                for d in (1, 2, 4):
                    v = v + jnp.concatenate(
                        [jnp.zeros((d, L), v.dtype), v[: 8 - d]], axis=0
                    )
                vs.append(v)
            offs = [carry[n : n + 1]]
            for u in range(_U - 1):
                offs.append(offs[-1] + vs[u][7:8, :])
            for u in range(_U):
                o_ref[n, pl.ds(k * rows + u * 8, 8), :] = vs[u] + offs[u]
            newc.append(offs[_U - 1] + vs[_U - 1][7:8, :])
        return jnp.concatenate(newc, axis=0)

    carry_ref[...] = jax.lax.fori_loop(0, _BS // rows, group, carry_ref[...])


def kernel(x):
    B, S, L = x.shape
    grid = (B // _NB, S // _BS)
    return pl.pallas_call(
        _scan_body,
        grid=grid,
        in_specs=[pl.BlockSpec((_NB, _BS, L), lambda i, j: (i, j, 0))],
        out_specs=pl.BlockSpec((_NB, _BS, L), lambda i, j: (i, j, 0)),
        out_shape=jax.ShapeDtypeStruct(x.shape, x.dtype),
        scratch_shapes=[pltpu.VMEM((_NB, L), jnp.float32)],
        compiler_params=pltpu.CompilerParams(
            dimension_semantics=("arbitrary", "arbitrary"),
        ),
    )(x)


# speedup vs baseline: 1.1872x; 1.0019x over previous
import jax
import jax.numpy as jnp
from jax.experimental import pallas as pl
from jax.experimental.pallas import tpu as pltpu

_BS = 256
_U = 8
_NB = 4


def _scan_body(x_ref, o_ref, carry_ref):
    j = pl.program_id(1)

    @pl.when(j == 0)
    def _():
        carry_ref[...] = jnp.zeros_like(carry_ref)

    L = x_ref.shape[2]
    rows = 8 * _U

    def group(k, carry):
        newc = []
        for n in range(_NB):
            vs = []
            for u in range(_U):
                v = x_ref[n, pl.ds(k * rows + u * 8, 8), :]
                for d in (1, 2, 4):
                    v = v + jnp.concatenate(
                        [jnp.zeros((d, L), v.dtype), v[: 8 - d]], axis=0
                    )
                vs.append(v)
            offs = [carry[n : n + 1]]
            for u in range(_U - 1):
                offs.append(offs[-1] + vs[u][7:8, :])
            for u in range(_U):
                o_ref[n, pl.ds(k * rows + u * 8, 8), :] = vs[u] + offs[u]
            newc.append(offs[_U - 1] + vs[_U - 1][7:8, :])
        return jnp.concatenate(newc, axis=0)

    carry_ref[...] = jax.lax.fori_loop(0, _BS // rows, group, carry_ref[...])


def kernel(x):
    B, S, L = x.shape
    grid = (B // _NB, S // _BS)
    return pl.pallas_call(
        _scan_body,
        grid=grid,
        in_specs=[pl.BlockSpec((_NB, _BS, L), lambda i, j: (i, j, 0))],
        out_specs=pl.BlockSpec((_NB, _BS, L), lambda i, j: (i, j, 0)),
        out_shape=jax.ShapeDtypeStruct(x.shape, x.dtype),
        scratch_shapes=[pltpu.VMEM((_NB, L), jnp.float32)],
        compiler_params=pltpu.CompilerParams(
            dimension_semantics=("arbitrary", "arbitrary"),
        ),
    )(x)
